# Initial kernel scaffold; baseline (speedup 1.0000x reference)
import jax
import jax.numpy as jnp
from jax.experimental import pallas as pl


def kernel(X, edge_index0, edge_index1, edge_weight0, edge_weight1, res_n_id0, res_n_id1, Wn1, We1, Q1, K1, Wa1, ba1, Wo1, bo1, Wn2, We2, Q2, K2, Wa2, ba2, Wo2, bo2):
    raise NotImplementedError("write your pallas kernel here")


# speedup vs baseline: 45.8573x; 45.8573x over previous
import functools

import jax
import jax.numpy as jnp
from jax import lax
from jax.experimental import pallas as pl
from jax.experimental.pallas import tpu as pltpu
from jax.experimental.pallas import tpu_sc as plsc

N = 10000
E = 320000
B = 4
C = 16
BC = B * C
F_IN = 128

NW = 32
NSUB = 16
CHUNK = 128
EPT = 10240
NCHUNK = EPT // CHUNK
PAD_ROWS = 16
ROWS_PER_SUB = 624

_f32 = jnp.float32
_i32 = jnp.int32


_R = 2000


def _prep_body(x_ref, wn_ref, wa8_ref, v_ref, a_ref):
    wn = wn_ref[...]
    vs = [jnp.dot(x_ref[b], wn, preferred_element_type=_f32) for b in range(B)]
    v64 = jnp.concatenate(vs, axis=-1)
    v_ref[...] = v64
    a_ref[...] = jnp.dot(v64, wa8_ref[...], preferred_element_type=_f32)


_prep = pl.pallas_call(
    _prep_body,
    grid=(N // _R,),
    in_specs=[
        pl.BlockSpec((B, _R, F_IN), lambda i: (0, i, 0)),
        pl.BlockSpec((F_IN, C), lambda i: (0, 0)),
        pl.BlockSpec((BC, 8), lambda i: (0, 0)),
    ],
    out_specs=[
        pl.BlockSpec((_R, BC), lambda i: (i, 0)),
        pl.BlockSpec((_R, 8), lambda i: (i, 0)),
    ],
    out_shape=[
        jax.ShapeDtypeStruct((N, BC), _f32),
        jax.ShapeDtypeStruct((N, 8), _f32),
    ],
)


def _mid_body(v_ref, agg_ref, wo_ref, bo_ref, wn2_ref, wa8_ref, v2_ref, a2_ref):
    v = v_ref[...]
    agg = agg_ref[0] + agg_ref[1]
    h = (v + jnp.dot(v, wo_ref[0:BC], preferred_element_type=_f32)
         + jnp.dot(agg, wo_ref[BC:2 * BC], preferred_element_type=_f32)
         + bo_ref[...])
    h = jnp.where(h >= 0, h, 0.01 * h)
    v2 = jnp.dot(h, wn2_ref[...], preferred_element_type=_f32)
    v2_ref[...] = v2
    a2_ref[...] = jnp.dot(v2, wa8_ref[...], preferred_element_type=_f32)


_mid = pl.pallas_call(
    _mid_body,
    grid=(N // _R,),
    in_specs=[
        pl.BlockSpec((_R, BC), lambda i: (i, 0)),
        pl.BlockSpec((2, _R, BC), lambda i: (0, i, 0)),
        pl.BlockSpec((2 * BC, BC), lambda i: (0, 0)),
        pl.BlockSpec((1, BC), lambda i: (0, 0)),
        pl.BlockSpec((BC, BC), lambda i: (0, 0)),
        pl.BlockSpec((BC, 8), lambda i: (0, 0)),
    ],
    out_specs=[
        pl.BlockSpec((_R, BC), lambda i: (i, 0)),
        pl.BlockSpec((_R, 8), lambda i: (i, 0)),
    ],
    out_shape=[
        jax.ShapeDtypeStruct((N, BC), _f32),
        jax.ShapeDtypeStruct((N, 8), _f32),
    ],
)


def _fin_body(v_ref, agg_ref, wo_ref, bo_ref, o_ref):
    v = v_ref[...]
    agg = agg_ref[0] + agg_ref[1]
    h = (v + jnp.dot(v, wo_ref[0:BC], preferred_element_type=_f32)
         + jnp.dot(agg, wo_ref[BC:2 * BC], preferred_element_type=_f32)
         + bo_ref[...])
    o_ref[...] = jnp.where(h >= 0, h, 0.01 * h)


_fin = pl.pallas_call(
    _fin_body,
    grid=(N // _R,),
    in_specs=[
        pl.BlockSpec((_R, BC), lambda i: (i, 0)),
        pl.BlockSpec((2, _R, BC), lambda i: (0, i, 0)),
        pl.BlockSpec((2 * BC, BC), lambda i: (0, 0)),
        pl.BlockSpec((1, BC), lambda i: (0, 0)),
    ],
    out_specs=pl.BlockSpec((_R, BC), lambda i: (i, 0)),
    out_shape=jax.ShapeDtypeStruct((N, BC), _f32),
)


_GATHER_1D = lax.GatherDimensionNumbers(
    offset_dims=(), collapsed_slice_dims=(0,), start_index_map=(0,))


def _vperm(x, idx):
    return lax.gather(x, idx[:, None], _GATHER_1D, (1,),
                      mode=lax.GatherScatterMode.PROMISE_IN_BOUNDS)


_mesh = plsc.VectorSubcoreMesh(core_axis_name="c", subcore_axis_name="s")


@functools.partial(
    pl.kernel,
    mesh=_mesh,
    compiler_params=pltpu.CompilerParams(
        needs_layout_passes=False, use_tc_tiling_on_sc=False),
    out_type=jax.ShapeDtypeStruct((2, N, BC), _f32),
    scratch_types=[
        pltpu.VMEM(((N + PAD_ROWS) * 8,), _f32),
        pltpu.VMEM((CHUNK, BC), _f32),
        pltpu.VMEM((CHUNK,), _i32),
        pltpu.VMEM((CHUNK,), _i32),
        pltpu.VMEM((CHUNK,), _f32),
        pltpu.VMEM((3, 16), _f32),
        pltpu.VMEM_SHARED((N + PAD_ROWS, BC), _f32),
        pltpu.SemaphoreType.DMA,
    ],
)
def _sc_edge_pass(vtab, atab, srcs, dsts, ws, params, out,
                  a_v, rows, sidx, didx, wbuf, pv, aggr, gsem):
    cid = lax.axis_index("c")
    sid = lax.axis_index("s")
    wid = sid * 2 + cid
    iota = lax.iota(_i32, 16)
    zeros16 = jnp.zeros((16,), _f32)

    pltpu.sync_copy(params, pv)
    pltpu.sync_copy(atab, a_v.at[pl.ds(0, N * 8)])

    def _zrow(r, _):
        for q in range(B):
            rows[r, pl.ds(q * 16, 16)] = zeros16
        return 0
    lax.fori_loop(0, CHUNK, _zrow, 0)
    z0 = sid * ROWS_PER_SUB
    for off, sz in ((0, 128), (128, 128), (256, 128), (384, 128), (512, 112)):
        pltpu.sync_copy(rows.at[pl.ds(0, sz)], aggr.at[pl.ds(z0 + off, sz)])

    @pl.when(sid == NSUB - 1)
    def _zero_tail():
        pltpu.sync_copy(rows.at[pl.ds(0, 32)],
                        aggr.at[pl.ds(NSUB * ROWS_PER_SUB, 32)])
    plsc.subcore_barrier()

    negwe = pv[0, :]
    kapv = pv[1, :]
    bav = pv[2, :]

    def _chunk(j, _):
        base = j * CHUNK
        pltpu.sync_copy(srcs.at[wid, pl.ds(base, CHUNK)], sidx)
        pltpu.sync_copy(dsts.at[wid, pl.ds(base, CHUNK)], didx)
        pltpu.sync_copy(ws.at[wid, pl.ds(base, CHUNK)], wbuf)
        pltpu.async_copy(vtab.at[sidx], rows, gsem).wait()

        def _group(gi, _):
            e0 = gi * 16
            ev = e0 + iota
            sv = plsc.load_gather(sidx, [ev]) * 8
            dv = plsc.load_gather(didx, [ev]) * 8
            wv = plsc.load_gather(wbuf, [ev])
            cw = kapv * wv + bav
            atts = []
            for bb in range(B):
                a_s = plsc.load_gather(a_v, [sv + (2 * bb)])
                a_d = plsc.load_gather(a_v, [dv + (2 * bb + 1)])
                atts.append(1.0 / (1.0 + jnp.exp(-(a_s + a_d + cw))))
            for i in range(16):
                li = jnp.full((16,), i, _i32)
                we = _vperm(wv, li)
                g = 1.0 / (1.0 + jnp.exp(we * negwe))
                ei = e0 + i
                for bb in range(B):
                    ab = _vperm(atts[bb], li)
                    t = ab * g
                    rows[ei, pl.ds(bb * 16, 16)] = rows[ei, pl.ds(bb * 16, 16)] * t
            return 0

        lax.fori_loop(0, CHUNK // 16, _group, 0)
        pltpu.sync_copy(rows, aggr.at[didx], add=True)
        return 0

    lax.fori_loop(0, NCHUNK, _chunk, 0)
    plsc.subcore_barrier()

    pltpu.sync_copy(
        aggr.at[pl.ds(sid * ROWS_PER_SUB, ROWS_PER_SUB)],
        out.at[cid, pl.ds(sid * ROWS_PER_SUB, ROWS_PER_SUB)])

    @pl.when(sid == NSUB - 1)
    def _write_tail():
        pltpu.sync_copy(
            aggr.at[pl.ds(NSUB * ROWS_PER_SUB, N - NSUB * ROWS_PER_SUB)],
            out.at[cid, pl.ds(NSUB * ROWS_PER_SUB, N - NSUB * ROWS_PER_SUB)])


def _fold_attention(We, Q, K, Wa, ba):
    qa = Q @ Wa[:C, 0]
    ka = K @ Wa[C:2 * C, 0]
    kappa = We[0] @ Wa[2 * C:, 0]
    wa8 = jnp.zeros((BC, 8), _f32)
    for b in range(B):
        wa8 = wa8.at[b * C:(b + 1) * C, 2 * b].set(qa)
        wa8 = wa8.at[b * C:(b + 1) * C, 2 * b + 1].set(ka)
    params = jnp.stack([
        -We[0],
        jnp.full((C,), kappa, _f32),
        jnp.full((C,), ba[0], _f32),
    ])
    return wa8, params


def _block_diag(W):
    z = jnp.zeros((BC, BC), _f32)
    for b in range(B):
        z = z.at[b * C:(b + 1) * C, b * C:(b + 1) * C].set(W)
    return z


def _fold_out(Wo, bo):
    wo_cat = jnp.concatenate([_block_diag(Wo[:C]), _block_diag(Wo[C:])], axis=0)
    bo64 = jnp.tile(bo, B)[None, :]
    return wo_cat, bo64


def _pack_edges(ei, w):
    ept0 = E // NW
    src = ei[0].astype(_i32).reshape(NW, ept0)
    dst = ei[1].astype(_i32).reshape(NW, ept0)
    wr = w.astype(_f32).reshape(NW, ept0)
    pad = EPT - ept0
    src = jnp.pad(src, ((0, 0), (0, pad)))
    dst = jnp.pad(dst, ((0, 0), (0, pad)), constant_values=N)
    wr = jnp.pad(wr, ((0, 0), (0, pad)))
    return src, dst, wr


def kernel(X, edge_index0, edge_index1, edge_weight0, edge_weight1,
           res_n_id0, res_n_id1, Wn1, We1, Q1, K1, Wa1, ba1, Wo1, bo1,
           Wn2, We2, Q2, K2, Wa2, ba2, Wo2, bo2):
    wa8_1, params1 = _fold_attention(We1, Q1, K1, Wa1, ba1)
    wa8_2, params2 = _fold_attention(We2, Q2, K2, Wa2, ba2)
    wo1_cat, bo1_64 = _fold_out(Wo1, bo1)
    wo2_cat, bo2_64 = _fold_out(Wo2, bo2)
    wn2_bd = _block_diag(Wn2)
    s0, d0, w0 = _pack_edges(edge_index0, edge_weight0)
    s1, d1, w1 = _pack_edges(edge_index1, edge_weight1)

    v1, a1 = _prep(X, Wn1, wa8_1)
    agg1 = _sc_edge_pass(v1, a1.reshape(N * 8), s0, d0, w0, params1)
    v2, a2 = _mid(v1, agg1, wo1_cat, bo1_64, wn2_bd, wa8_2)
    agg2 = _sc_edge_pass(v2, a2.reshape(N * 8), s1, d1, w1, params2)
    out64 = _fin(v2, agg2, wo2_cat, bo2_64)
    return out64.reshape(N, B, C).transpose(1, 0, 2)
